# Initial kernel scaffold; baseline (speedup 1.0000x reference)
import jax
import jax.numpy as jnp
from jax.experimental import pallas as pl


def kernel(xyz, points, W1, b1, W2, b2, W3, b3):
    raise NotImplementedError("write your pallas kernel here")


# speedup vs baseline: 18.5497x; 18.5497x over previous
import functools

import jax
import jax.numpy as jnp
from jax import lax
from jax.experimental import pallas as pl
from jax.experimental.pallas import tpu as pltpu
from jax.experimental.pallas import tpu_sc as plsc

B = 16
N = 4096
S = 1024
K = 32
R2 = 0.2 ** 2
CIN = 35
CPAD = 48
SBLK = 256
MBLK = 128

_INTERPRET = False


def _fps_body(x_ref, out_ref):
    xs = x_ref[:, 0, :]
    ys = x_ref[:, 1, :]
    zs = x_ref[:, 2, :]
    iota = lax.broadcasted_iota(jnp.int32, (B, N), 1)
    iota_s = lax.broadcasted_iota(jnp.int32, (B, 3, S), 2)

    def body(i, carry):
        distance, far, nxc = carry
        sel = iota == far
        cx = jnp.sum(jnp.where(sel, xs, 0.0), axis=1, keepdims=True)
        cy = jnp.sum(jnp.where(sel, ys, 0.0), axis=1, keepdims=True)
        cz = jnp.sum(jnp.where(sel, zs, 0.0), axis=1, keepdims=True)
        cvals = jnp.concatenate([cx, cy, cz], axis=1)[:, :, None]
        nxc = jnp.where(iota_s == i, cvals, nxc)
        dx = xs - cx
        dy = ys - cy
        dz = zs - cz
        dist = (dx * dx + dz * dz) + dy * dy
        distance = jnp.minimum(distance, dist)
        m = jnp.max(distance, axis=1, keepdims=True)
        far_new = jnp.min(
            jnp.where(distance == m, iota, N), axis=1, keepdims=True)
        return distance, far_new, nxc

    distance0 = jnp.full((B, N), 1e10, dtype=jnp.float32)
    far0 = jnp.zeros((B, 1), dtype=jnp.int32)
    nxc0 = jnp.zeros((B, 3, S), dtype=jnp.float32)
    _, _, nxc = lax.fori_loop(0, S, body, (distance0, far0, nxc0))
    out_ref[...] = nxc


def _fps(xt):
    return pl.pallas_call(
        _fps_body,
        out_shape=jax.ShapeDtypeStruct((B, 3, S), jnp.float32),
        interpret=_INTERPRET,
    )(xt)


def _bq_body(x_ref, nx_ref, idx_ref):
    cb = pl.program_id(1)
    xr = x_ref[0]
    cc = nx_ref[0, :, pl.ds(cb * SBLK, SBLK)]
    xr2 = xr * xr
    p_norm = (xr2[0:1] + xr2[1:2]) + xr2[2:3]
    cc2 = cc * cc
    c_norm = (cc2[0:1] + cc2[1:2]) + cc2[2:3]
    dots = jnp.dot(cc.T, xr, preferred_element_type=jnp.float32)
    sqr = c_norm.T + p_norm - 2.0 * dots

    iota = lax.broadcasted_iota(jnp.int32, (SBLK, N), 1)
    cur = jnp.where(sqr <= R2, iota, N)
    col0 = jnp.min(cur, axis=1, keepdims=True)
    cur = jnp.where(cur == col0, N, cur)
    cols = [col0]
    for _ in range(K - 1):
        m = jnp.min(cur, axis=1, keepdims=True)
        cur = jnp.where(cur == m, N, cur)
        cols.append(jnp.where(m == N, col0, m))
    idx_ref[0] = jnp.concatenate(cols, axis=1)


def _ballquery(xt, nx):
    return pl.pallas_call(
        _bq_body,
        grid=(B, S // SBLK),
        in_specs=[
            pl.BlockSpec((1, 3, N), lambda b, cb: (b, 0, 0)),
            pl.BlockSpec((1, 3, S), lambda b, cb: (b, 0, 0)),
        ],
        out_specs=pl.BlockSpec((1, SBLK, K), lambda b, cb: (b, cb, 0)),
        out_shape=jax.ShapeDtypeStruct((B, S, K), jnp.int32),
        interpret=_INTERPRET,
    )(xt, nx)


_NW = 32
_CHUNK = 1024
_ROWS = B * S * K


def _sc_gather(table, idxf):
    mesh = plsc.VectorSubcoreMesh(core_axis_name="c", subcore_axis_name="s")
    rows_per_w = _ROWS // _NW

    @functools.partial(
        pl.kernel,
        mesh=mesh,
        out_type=jax.ShapeDtypeStruct((_ROWS, CPAD), jnp.float32),
        scratch_types=[
            pltpu.VMEM((_CHUNK,), jnp.int32),
            pltpu.VMEM((_CHUNK, CPAD), jnp.float32),
            pltpu.SemaphoreType.DMA,
        ],
        compiler_params=pltpu.CompilerParams(use_tc_tiling_on_sc=False),
    )
    def k(table_hbm, idx_hbm, out_hbm, idx_v, rows_v, sem):
        wid = lax.axis_index("s") * 2 + lax.axis_index("c")
        base = wid * rows_per_w
        for c in range(rows_per_w // _CHUNK):
            off = base + c * _CHUNK
            pltpu.sync_copy(idx_hbm.at[pl.ds(off, _CHUNK)], idx_v)
            pltpu.async_copy(table_hbm.at[idx_v], rows_v, sem).wait()
            pltpu.sync_copy(rows_v, out_hbm.at[pl.ds(off, _CHUNK)])

    return k(table, idxf)


def _mlp_body(g_ref, nx_ref, w1_ref, w2_ref, w3_ref,
              b1_ref, b2_ref, b3_ref, out_ref):
    cb = pl.program_id(1)
    x = g_ref[...]
    w1 = w1_ref[...]
    cc = nx_ref[0, :, pl.ds(cb * MBLK, MBLK)]
    shift = jnp.dot(cc.T, w1[0:3], preferred_element_type=jnp.float32)
    h = jnp.dot(x, w1, preferred_element_type=jnp.float32) + b1_ref[...]
    h = h.reshape(MBLK, K, 32) - shift[:, None, :]
    h = jnp.maximum(h, 0.0).reshape(MBLK * K, 32)
    h = jnp.dot(h, w2_ref[...], preferred_element_type=jnp.float32) + b2_ref[...]
    h = jnp.maximum(h, 0.0)
    h = jnp.dot(h, w3_ref[...], preferred_element_type=jnp.float32) + b3_ref[...]
    h = jnp.maximum(h, 0.0)
    out_ref[0] = jnp.max(h.reshape(MBLK, K, 64), axis=1)


def _mlp(g, nx, w1p, w2, w3, b1, b2, b3):
    nblk = S // MBLK
    return pl.pallas_call(
        _mlp_body,
        grid=(B, nblk),
        in_specs=[
            pl.BlockSpec((MBLK * K, CPAD), lambda b, cb: (b * nblk + cb, 0)),
            pl.BlockSpec((1, 3, S), lambda b, cb: (b, 0, 0)),
            pl.BlockSpec((CPAD, 32), lambda b, cb: (0, 0)),
            pl.BlockSpec((32, 32), lambda b, cb: (0, 0)),
            pl.BlockSpec((32, 64), lambda b, cb: (0, 0)),
            pl.BlockSpec((1, 32), lambda b, cb: (0, 0)),
            pl.BlockSpec((1, 32), lambda b, cb: (0, 0)),
            pl.BlockSpec((1, 64), lambda b, cb: (0, 0)),
        ],
        out_specs=pl.BlockSpec((1, MBLK, 64), lambda b, cb: (b, cb, 0)),
        out_shape=jax.ShapeDtypeStruct((B, S, 64), jnp.float32),
        interpret=_INTERPRET,
    )(g, nx, w1p, w2, w3, b1, b2, b3)


def kernel(xyz, points, W1, b1, W2, b2, W3, b3):
    xt = jnp.transpose(xyz, (0, 2, 1))
    nx = _fps(xt)
    idx = _ballquery(xt, nx)

    pad = jnp.zeros((B, N, CPAD - CIN), dtype=jnp.float32)
    table = jnp.concatenate([xyz, points, pad], axis=-1).reshape(B * N, CPAD)
    idxf = (idx + (jnp.arange(B, dtype=jnp.int32) * N)[:, None, None])
    idxf = idxf.reshape(_ROWS)
    g = _sc_gather(table, idxf)

    w1p = jnp.concatenate(
        [W1, jnp.zeros((CPAD - CIN, 32), dtype=jnp.float32)], axis=0)
    out = _mlp(g, nx, w1p, W2, W3,
               b1.reshape(1, 32), b2.reshape(1, 32), b3.reshape(1, 64))
    new_xyz = jnp.transpose(nx, (0, 2, 1))
    return (new_xyz, out)
